# baseline (speedup 1.0000x reference)
import functools

import jax
import jax.numpy as jnp
from jax import lax
from jax.experimental import pallas as pl
from jax.experimental.pallas import tpu as pltpu
from jax.experimental.pallas import tpu_sc as plsc

_N = 10000
_D = 128
_NBLK = 79
_NROWS = _NBLK * 128
_NC = 2
_NS = 16
_NW = _NC * _NS
_C = 128
_DEGR = 80


_RPT = 16
_CA = 64


def _sc_aggregate(table, src2d, dst2d, nround):
    mesh = plsc.VectorSubcoreMesh(core_axis_name="c", subcore_axis_name="s")
    rows_per_tile = _NROWS // _NS

    out_type = jax.ShapeDtypeStruct((_NC, _NROWS, 128), jnp.float32)

    scratch = dict(
        src_v=pltpu.VMEM((_RPT, _CA), jnp.int32),
        dst_v=pltpu.VMEM((_RPT, _CA), jnp.int32),
        rows0=pltpu.VMEM((_CA, 128), jnp.float32),
        rows1=pltpu.VMEM((_CA, 128), jnp.float32),
        acc_sh=pltpu.VMEM_SHARED((_NROWS, 128), jnp.float32),
        gsem0=pltpu.SemaphoreType.DMA,
        gsem1=pltpu.SemaphoreType.DMA,
        ssem0=pltpu.SemaphoreType.DMA,
        ssem1=pltpu.SemaphoreType.DMA,
    )

    def body(table_hbm, src_hbm, dst_hbm, acc_out, **sc):
        c = lax.axis_index("c")
        s = lax.axis_index("s")
        wid = s * _NC + c
        rows = (sc["rows0"], sc["rows1"])
        gsem = (sc["gsem0"], sc["gsem1"])
        ssem = (sc["ssem0"], sc["ssem1"])

        zero16 = jnp.zeros((16,), jnp.float32)

        def zfill(i, carry):
            for j in range(8):
                sc["rows0"][i, pl.ds(j * 16, 16)] = zero16
            return carry
        lax.fori_loop(0, _CA, zfill, 0)

        for k in range(9):
            pltpu.sync_copy(
                sc["rows0"],
                sc["acc_sh"].at[pl.ds(s * rows_per_tile + k * _CA, _CA)])
        pltpu.sync_copy(
            sc["rows0"].at[pl.ds(0, rows_per_tile - 9 * _CA)],
            sc["acc_sh"].at[pl.ds(s * rows_per_tile + 9 * _CA,
                                  rows_per_tile - 9 * _CA)])

        plsc.subcore_barrier()

        def round_body(r, carry):
            pltpu.sync_copy(src_hbm.at[wid, pl.ds(r * _RPT, _RPT)],
                            sc["src_v"])
            pltpu.sync_copy(dst_hbm.at[wid, pl.ds(r * _RPT, _RPT)],
                            sc["dst_v"])

            pltpu.async_copy(table_hbm.at[sc["src_v"].at[0]],
                             rows[0], gsem[0])

            def step2(i2, carry2):
                for p in range(2):
                    i = i2 * 2 + p
                    q = 1 - p
                    pltpu.make_async_copy(
                        table_hbm.at[sc["src_v"].at[i]], rows[p],
                        gsem[p]).wait()
                    pltpu.async_copy(rows[p],
                                     sc["acc_sh"].at[sc["dst_v"].at[i]],
                                     ssem[p], add=True)
                    @pl.when(jnp.logical_or(i2 > 0, p > 0))
                    def _():
                        pltpu.make_async_copy(
                            rows[q],
                            sc["acc_sh"].at[sc["dst_v"].at[i - 1]],
                            ssem[q]).wait()

                    @pl.when(i < _RPT - 1)
                    def _():
                        pltpu.async_copy(
                            table_hbm.at[sc["src_v"].at[i + 1]],
                            rows[q], gsem[q])
                return carry2
            lax.fori_loop(0, _RPT // 2, step2, 0)

            pltpu.make_async_copy(
                rows[1], sc["acc_sh"].at[sc["dst_v"].at[_RPT - 1]],
                ssem[1]).wait()
            return carry
        lax.fori_loop(0, nround, round_body, 0)

        plsc.subcore_barrier()

        pltpu.sync_copy(
            sc["acc_sh"].at[pl.ds(s * rows_per_tile, rows_per_tile)],
            acc_out.at[c, pl.ds(s * rows_per_tile, rows_per_tile)])

    return pl.kernel(body, out_type=out_type, mesh=mesh,
                     scratch_types=scratch)(table, src2d, dst2d)


def _sc_degree(dst2d, nround):
    mesh = plsc.VectorSubcoreMesh(core_axis_name="c", subcore_axis_name="s")
    rows_per_tile = _NROWS // _NS

    out_type = jax.ShapeDtypeStruct((_NC, _NROWS, 128), jnp.float32)

    scratch = dict(
        dst_v=pltpu.VMEM((_RPT, _C), jnp.int32),
        ones_v=pltpu.VMEM((_C, 128), jnp.float32),
        deg_sh=pltpu.VMEM_SHARED((_NROWS, 128), jnp.float32),
        ssem=pltpu.SemaphoreType.DMA,
    )

    def body(dst_hbm, deg_out, **sc):
        c = lax.axis_index("c")
        s = lax.axis_index("s")
        wid = s * _NC + c

        zero16 = jnp.zeros((16,), jnp.float32)
        ones16 = jnp.ones((16,), jnp.float32)

        def fill_zero(i, carry):
            for j in range(8):
                sc["ones_v"][i, pl.ds(j * 16, 16)] = zero16
            return carry
        lax.fori_loop(0, _C, fill_zero, 0)

        for k in range(4):
            pltpu.sync_copy(
                sc["ones_v"],
                sc["deg_sh"].at[pl.ds(s * rows_per_tile + k * 128, 128)])
        pltpu.sync_copy(
            sc["ones_v"].at[pl.ds(0, 120)],
            sc["deg_sh"].at[pl.ds(s * rows_per_tile + 512, 120)])

        def fill_ones(i, carry):
            for j in range(8):
                sc["ones_v"][i, pl.ds(j * 16, 16)] = ones16
            return carry
        lax.fori_loop(0, _C, fill_ones, 0)

        plsc.subcore_barrier()

        def round_body(r, carry):
            pltpu.sync_copy(dst_hbm.at[wid, pl.ds(r * _RPT, _RPT)],
                            sc["dst_v"])

            def fire(i, carry2):
                pltpu.async_copy(sc["ones_v"],
                                 sc["deg_sh"].at[sc["dst_v"].at[i]],
                                 sc["ssem"], add=True)
                return carry2
            lax.fori_loop(0, _RPT, fire, 0)

            def drain(i, carry2):
                pltpu.make_async_copy(sc["ones_v"],
                                      sc["deg_sh"].at[sc["dst_v"].at[i]],
                                      sc["ssem"]).wait()
                return carry2
            lax.fori_loop(0, _RPT, drain, 0)
            return carry
        lax.fori_loop(0, nround, round_body, 0)

        plsc.subcore_barrier()

        pltpu.sync_copy(
            sc["deg_sh"].at[pl.ds(s * rows_per_tile, rows_per_tile)],
            deg_out.at[c, pl.ds(s * rows_per_tile, rows_per_tile)])

    return pl.kernel(body, out_type=out_type, mesh=mesh,
                     scratch_types=scratch)(dst2d)


def _tc_proj(x, w_l, w_r, b):
    def body(x_ref, wl_ref, wr_ref, b_ref, y_ref, z_ref):
        xb = x_ref[...]
        y_ref[...] = jnp.dot(xb, wl_ref[...],
                             preferred_element_type=jnp.float32)
        z_ref[...] = jnp.dot(xb, wr_ref[...],
                             preferred_element_type=jnp.float32) + b_ref[...]

    return pl.pallas_call(
        body,
        grid=(_NBLK,),
        in_specs=[
            pl.BlockSpec((128, 128), lambda i: (i, 0)),
            pl.BlockSpec((128, 128), lambda i: (0, 0)),
            pl.BlockSpec((128, 128), lambda i: (0, 0)),
            pl.BlockSpec((1, 128), lambda i: (0, 0)),
        ],
        out_specs=[pl.BlockSpec((128, 128), lambda i: (i, 0))] * 2,
        out_shape=[jax.ShapeDtypeStruct((_NROWS, 128), jnp.float32)] * 2,
    )(x, w_l, w_r, b.reshape(1, 128))


def _tc_combine_proj(acc, deg, z1, w_l, w_r, b):
    def body(acc_ref, deg_ref, z1_ref, wl_ref, wr_ref, b_ref,
             y2_ref, z2_ref, rdeg_ref):
        a = acc_ref[0] + acc_ref[1]
        d = (deg_ref[0] + deg_ref[1])[:, 0:1]
        r = 1.0 / jnp.maximum(d, 1.0)
        h = jnp.maximum(a * r + z1_ref[...], 0.0)
        y2_ref[...] = jnp.dot(h, wl_ref[...],
                              preferred_element_type=jnp.float32)
        z2_ref[...] = jnp.dot(h, wr_ref[...],
                              preferred_element_type=jnp.float32) + b_ref[...]
        rdeg_ref[...] = r

    return pl.pallas_call(
        body,
        grid=(_NBLK,),
        in_specs=[
            pl.BlockSpec((_NC, 128, 128), lambda i: (0, i, 0)),
            pl.BlockSpec((_NC, 128, 128), lambda i: (0, i, 0)),
            pl.BlockSpec((128, 128), lambda i: (i, 0)),
            pl.BlockSpec((128, 128), lambda i: (0, 0)),
            pl.BlockSpec((128, 128), lambda i: (0, 0)),
            pl.BlockSpec((1, 128), lambda i: (0, 0)),
        ],
        out_specs=[
            pl.BlockSpec((128, 128), lambda i: (i, 0)),
            pl.BlockSpec((128, 128), lambda i: (i, 0)),
            pl.BlockSpec((128, 1), lambda i: (i, 0)),
        ],
        out_shape=[
            jax.ShapeDtypeStruct((_NROWS, 128), jnp.float32),
            jax.ShapeDtypeStruct((_NROWS, 128), jnp.float32),
            jax.ShapeDtypeStruct((_NROWS, 1), jnp.float32),
        ],
    )(acc, deg, z1, w_l, w_r, b.reshape(1, 128))


def _tc_final(acc, rdeg, z2):
    def body(acc_ref, rdeg_ref, z2_ref, o_ref):
        a = acc_ref[0] + acc_ref[1]
        o_ref[...] = a * rdeg_ref[...] + z2_ref[...]

    return pl.pallas_call(
        body,
        grid=(_NBLK,),
        in_specs=[
            pl.BlockSpec((_NC, 128, 128), lambda i: (0, i, 0)),
            pl.BlockSpec((128, 1), lambda i: (i, 0)),
            pl.BlockSpec((128, 128), lambda i: (i, 0)),
        ],
        out_specs=pl.BlockSpec((128, 128), lambda i: (i, 0)),
        out_shape=jax.ShapeDtypeStruct((_NROWS, 128), jnp.float32),
    )(acc, rdeg, z2)


def kernel(x, edge_index, W1_l, W1_r, b1, W2_l, W2_r, b2):
    E = edge_index.shape[1]
    epr = _NW * _C * _RPT
    e_pad = ((E + epr - 1) // epr) * epr
    nround_d = e_pad // epr
    nround = e_pad // (_NW * _CA * _RPT)

    src = jnp.pad(edge_index[0], (0, e_pad - E))
    dst = jnp.pad(edge_index[1], (0, e_pad - E), constant_values=_N)
    src2d = src.reshape(_NW, nround * _RPT, _CA)
    dst2d = dst.reshape(_NW, nround * _RPT, _CA)
    dst2deg = dst.reshape(_NW, nround_d * _RPT, _C)

    xp = jnp.pad(x, ((0, _NROWS - _N), (0, 0)))

    y1, z1 = _tc_proj(xp, W1_l, W1_r, b1)
    acc1 = _sc_aggregate(y1, src2d, dst2d, nround)
    deg = _sc_degree(dst2deg, nround_d)
    y2, z2, rdeg = _tc_combine_proj(acc1, deg, z1, W2_l, W2_r, b2)

    acc2 = _sc_aggregate(y2, src2d, dst2d, nround)
    out = _tc_final(acc2, rdeg, z2)
    return out[:_N]


# speedup vs baseline: 1.2243x; 1.2243x over previous
import functools

import jax
import jax.numpy as jnp
from jax import lax
from jax.experimental import pallas as pl
from jax.experimental.pallas import tpu as pltpu
from jax.experimental.pallas import tpu_sc as plsc

_N = 10000
_D = 128
_NBLK = 79
_NROWS = _NBLK * 128
_NC = 2
_NS = 16
_NW = _NC * _NS
_C = 128
_DEGR = 80


_RPT = 16


_NBUF = 2


def _sc_aggregate(table, src2d, dst2d, nround):
    mesh = plsc.VectorSubcoreMesh(core_axis_name="c", subcore_axis_name="s")
    rows_per_tile = _NROWS // _NS

    out_type = jax.ShapeDtypeStruct((_NC, _NROWS, 128), jnp.float32)

    scratch = dict(
        src_v=pltpu.VMEM((_RPT, _C), jnp.int32),
        dst_v=pltpu.VMEM((_RPT, _C), jnp.int32),
        acc_sh=pltpu.VMEM_SHARED((_NROWS, 128), jnp.float32),
        **{f"rows{b}": pltpu.VMEM((_C, 128), jnp.float32)
           for b in range(_NBUF)},
        **{f"gsem{b}": pltpu.SemaphoreType.DMA for b in range(_NBUF)},
        **{f"ssem{b}": pltpu.SemaphoreType.DMA for b in range(_NBUF)},
    )

    def body(table_hbm, src_hbm, dst_hbm, acc_out, **sc):
        c = lax.axis_index("c")
        s = lax.axis_index("s")
        wid = s * _NC + c
        rows = tuple(sc[f"rows{b}"] for b in range(_NBUF))
        gsem = tuple(sc[f"gsem{b}"] for b in range(_NBUF))
        ssem = tuple(sc[f"ssem{b}"] for b in range(_NBUF))

        zero16 = jnp.zeros((16,), jnp.float32)

        def zfill(i, carry):
            for j in range(8):
                sc["rows0"][i, pl.ds(j * 16, 16)] = zero16
            return carry
        lax.fori_loop(0, _C, zfill, 0)

        for k in range(4):
            pltpu.sync_copy(
                sc["rows0"],
                sc["acc_sh"].at[pl.ds(s * rows_per_tile + k * _C, _C)])
        pltpu.sync_copy(
            sc["rows0"].at[pl.ds(0, rows_per_tile - 4 * _C)],
            sc["acc_sh"].at[pl.ds(s * rows_per_tile + 4 * _C,
                                  rows_per_tile - 4 * _C)])

        plsc.subcore_barrier()

        def gather(i, b):
            return pltpu.async_copy(table_hbm.at[sc["src_v"].at[i]],
                                    rows[b], gsem[b])

        def scatter(i, b):
            return pltpu.async_copy(rows[b],
                                    sc["acc_sh"].at[sc["dst_v"].at[i]],
                                    ssem[b], add=True)

        def round_body(r, carry):
            pltpu.sync_copy(src_hbm.at[wid, pl.ds(r * _RPT, _RPT)],
                            sc["src_v"])
            pltpu.sync_copy(dst_hbm.at[wid, pl.ds(r * _RPT, _RPT)],
                            sc["dst_v"])

            for i in range(_NBUF - 1):
                gather(i, i)
            for i in range(_RPT):
                p = i % _NBUF
                pltpu.make_async_copy(table_hbm.at[sc["src_v"].at[i]],
                                      rows[p], gsem[p]).wait()
                scatter(i, p)
                if i > 0:
                    q = (i - 1) % _NBUF
                    pltpu.make_async_copy(
                        rows[q], sc["acc_sh"].at[sc["dst_v"].at[i - 1]],
                        ssem[q]).wait()
                if i + _NBUF - 1 < _RPT:
                    gather(i + _NBUF - 1, (i + _NBUF - 1) % _NBUF)
            pltpu.make_async_copy(
                rows[(_RPT - 1) % _NBUF],
                sc["acc_sh"].at[sc["dst_v"].at[_RPT - 1]],
                ssem[(_RPT - 1) % _NBUF]).wait()
            return carry
        lax.fori_loop(0, nround, round_body, 0)

        plsc.subcore_barrier()

        pltpu.sync_copy(
            sc["acc_sh"].at[pl.ds(s * rows_per_tile, rows_per_tile)],
            acc_out.at[c, pl.ds(s * rows_per_tile, rows_per_tile)])

    return pl.kernel(body, out_type=out_type, mesh=mesh,
                     scratch_types=scratch)(table, src2d, dst2d)


def _sc_degree(dst2d, nround):
    mesh = plsc.VectorSubcoreMesh(core_axis_name="c", subcore_axis_name="s")
    rows_per_tile = _NROWS // _NS

    out_type = jax.ShapeDtypeStruct((_NC, _NROWS, 128), jnp.float32)

    scratch = dict(
        dst_v=pltpu.VMEM((_RPT, _C), jnp.int32),
        ones_v=pltpu.VMEM((_C, 128), jnp.float32),
        deg_sh=pltpu.VMEM_SHARED((_NROWS, 128), jnp.float32),
        ssem=pltpu.SemaphoreType.DMA,
    )

    def body(dst_hbm, deg_out, **sc):
        c = lax.axis_index("c")
        s = lax.axis_index("s")
        wid = s * _NC + c

        zero16 = jnp.zeros((16,), jnp.float32)
        ones16 = jnp.ones((16,), jnp.float32)

        def fill_zero(i, carry):
            for j in range(8):
                sc["ones_v"][i, pl.ds(j * 16, 16)] = zero16
            return carry
        lax.fori_loop(0, _C, fill_zero, 0)

        for k in range(4):
            pltpu.sync_copy(
                sc["ones_v"],
                sc["deg_sh"].at[pl.ds(s * rows_per_tile + k * 128, 128)])
        pltpu.sync_copy(
            sc["ones_v"].at[pl.ds(0, 120)],
            sc["deg_sh"].at[pl.ds(s * rows_per_tile + 512, 120)])

        def fill_ones(i, carry):
            for j in range(8):
                sc["ones_v"][i, pl.ds(j * 16, 16)] = ones16
            return carry
        lax.fori_loop(0, _C, fill_ones, 0)

        plsc.subcore_barrier()

        def round_body(r, carry):
            pltpu.sync_copy(dst_hbm.at[wid, pl.ds(r * _RPT, _RPT)],
                            sc["dst_v"])

            def fire(i, carry2):
                pltpu.async_copy(sc["ones_v"],
                                 sc["deg_sh"].at[sc["dst_v"].at[i]],
                                 sc["ssem"], add=True)
                return carry2
            lax.fori_loop(0, _RPT, fire, 0)

            def drain(i, carry2):
                pltpu.make_async_copy(sc["ones_v"],
                                      sc["deg_sh"].at[sc["dst_v"].at[i]],
                                      sc["ssem"]).wait()
                return carry2
            lax.fori_loop(0, _RPT, drain, 0)
            return carry
        lax.fori_loop(0, nround, round_body, 0)

        plsc.subcore_barrier()

        pltpu.sync_copy(
            sc["deg_sh"].at[pl.ds(s * rows_per_tile, rows_per_tile)],
            deg_out.at[c, pl.ds(s * rows_per_tile, rows_per_tile)])

    return pl.kernel(body, out_type=out_type, mesh=mesh,
                     scratch_types=scratch)(dst2d)


def _tc_proj(x, w_l, w_r, b):
    def body(x_ref, wl_ref, wr_ref, b_ref, y_ref, z_ref):
        xb = x_ref[...]
        y_ref[...] = jnp.dot(xb, wl_ref[...],
                             preferred_element_type=jnp.float32)
        z_ref[...] = jnp.dot(xb, wr_ref[...],
                             preferred_element_type=jnp.float32) + b_ref[...]

    return pl.pallas_call(
        body,
        grid=(_NBLK,),
        in_specs=[
            pl.BlockSpec((128, 128), lambda i: (i, 0)),
            pl.BlockSpec((128, 128), lambda i: (0, 0)),
            pl.BlockSpec((128, 128), lambda i: (0, 0)),
            pl.BlockSpec((1, 128), lambda i: (0, 0)),
        ],
        out_specs=[pl.BlockSpec((128, 128), lambda i: (i, 0))] * 2,
        out_shape=[jax.ShapeDtypeStruct((_NROWS, 128), jnp.float32)] * 2,
    )(x, w_l, w_r, b.reshape(1, 128))


def _tc_combine_proj(acc, deg, z1, w_l, w_r, b):
    def body(acc_ref, deg_ref, z1_ref, wl_ref, wr_ref, b_ref,
             y2_ref, z2_ref, rdeg_ref):
        a = acc_ref[0] + acc_ref[1]
        d = (deg_ref[0] + deg_ref[1])[:, 0:1]
        r = 1.0 / jnp.maximum(d, 1.0)
        h = jnp.maximum(a * r + z1_ref[...], 0.0)
        y2_ref[...] = jnp.dot(h, wl_ref[...],
                              preferred_element_type=jnp.float32)
        z2_ref[...] = jnp.dot(h, wr_ref[...],
                              preferred_element_type=jnp.float32) + b_ref[...]
        rdeg_ref[...] = r

    return pl.pallas_call(
        body,
        grid=(_NBLK,),
        in_specs=[
            pl.BlockSpec((_NC, 128, 128), lambda i: (0, i, 0)),
            pl.BlockSpec((_NC, 128, 128), lambda i: (0, i, 0)),
            pl.BlockSpec((128, 128), lambda i: (i, 0)),
            pl.BlockSpec((128, 128), lambda i: (0, 0)),
            pl.BlockSpec((128, 128), lambda i: (0, 0)),
            pl.BlockSpec((1, 128), lambda i: (0, 0)),
        ],
        out_specs=[
            pl.BlockSpec((128, 128), lambda i: (i, 0)),
            pl.BlockSpec((128, 128), lambda i: (i, 0)),
            pl.BlockSpec((128, 1), lambda i: (i, 0)),
        ],
        out_shape=[
            jax.ShapeDtypeStruct((_NROWS, 128), jnp.float32),
            jax.ShapeDtypeStruct((_NROWS, 128), jnp.float32),
            jax.ShapeDtypeStruct((_NROWS, 1), jnp.float32),
        ],
    )(acc, deg, z1, w_l, w_r, b.reshape(1, 128))


def _tc_final(acc, rdeg, z2):
    def body(acc_ref, rdeg_ref, z2_ref, o_ref):
        a = acc_ref[0] + acc_ref[1]
        o_ref[...] = a * rdeg_ref[...] + z2_ref[...]

    return pl.pallas_call(
        body,
        grid=(_NBLK,),
        in_specs=[
            pl.BlockSpec((_NC, 128, 128), lambda i: (0, i, 0)),
            pl.BlockSpec((128, 1), lambda i: (i, 0)),
            pl.BlockSpec((128, 128), lambda i: (i, 0)),
        ],
        out_specs=pl.BlockSpec((128, 128), lambda i: (i, 0)),
        out_shape=jax.ShapeDtypeStruct((_NROWS, 128), jnp.float32),
    )(acc, rdeg, z2)


def kernel(x, edge_index, W1_l, W1_r, b1, W2_l, W2_r, b2):
    E = edge_index.shape[1]
    epr = _NW * _C * _RPT
    e_pad = ((E + epr - 1) // epr) * epr
    nround_d = e_pad // epr
    nround = nround_d

    src = jnp.pad(edge_index[0], (0, e_pad - E))
    dst = jnp.pad(edge_index[1], (0, e_pad - E), constant_values=_N)
    src2d = src.reshape(_NW, nround * _RPT, _C)
    dst2d = dst.reshape(_NW, nround * _RPT, _C)
    dst2deg = dst2d

    xp = jnp.pad(x, ((0, _NROWS - _N), (0, 0)))

    y1, z1 = _tc_proj(xp, W1_l, W1_r, b1)
    acc1 = _sc_aggregate(y1, src2d, dst2d, nround)
    deg = _sc_degree(dst2deg, nround_d)
    y2, z2, rdeg = _tc_combine_proj(acc1, deg, z1, W2_l, W2_r, b2)

    acc2 = _sc_aggregate(y2, src2d, dst2d, nround)
    out = _tc_final(acc2, rdeg, z2)
    return out[:_N]


# speedup vs baseline: 1.2629x; 1.0315x over previous
import functools

import jax
import jax.numpy as jnp
from jax import lax
from jax.experimental import pallas as pl
from jax.experimental.pallas import tpu as pltpu
from jax.experimental.pallas import tpu_sc as plsc

_N = 10000
_D = 128
_NBLK = 79
_NROWS = _NBLK * 128
_NC = 2
_NS = 16
_NW = _NC * _NS
_C = 128
_DEGR = 80


_RPT = 16
_NBUF = 2


def _sc_aggregate(table, src2d, dst2d, nround):
    mesh = plsc.VectorSubcoreMesh(core_axis_name="c", subcore_axis_name="s")
    rows_per_tile = _NROWS // _NS

    out_type = jax.ShapeDtypeStruct((_NC, _NROWS, 128), jnp.float32)

    scratch = dict(
        src_v=pltpu.VMEM((_RPT, _C), jnp.int32),
        dst_v=pltpu.VMEM((_RPT, _C), jnp.int32),
        acc_sh=pltpu.VMEM_SHARED((_NROWS, 128), jnp.float32),
        **{f"rows{b}": pltpu.VMEM((_C, 128), jnp.float32)
           for b in range(_NBUF)},
        **{f"gsem{b}": pltpu.SemaphoreType.DMA for b in range(_NBUF)},
        **{f"ssem{b}": pltpu.SemaphoreType.DMA for b in range(_NBUF)},
    )

    def body(table_hbm, src_hbm, dst_hbm, acc_out, **sc):
        c = lax.axis_index("c")
        s = lax.axis_index("s")
        wid = s * _NC + c

        zero16 = jnp.zeros((16,), jnp.float32)

        def zfill(i, carry):
            for j in range(8):
                sc["rows0"][i, pl.ds(j * 16, 16)] = zero16
            return carry
        lax.fori_loop(0, _C, zfill, 0)

        for k in range(4):
            pltpu.sync_copy(
                sc["rows0"],
                sc["acc_sh"].at[pl.ds(s * rows_per_tile + k * _C, _C)])
        pltpu.sync_copy(
            sc["rows0"].at[pl.ds(0, rows_per_tile - 4 * _C)],
            sc["acc_sh"].at[pl.ds(s * rows_per_tile + 4 * _C,
                                  rows_per_tile - 4 * _C)])

        plsc.subcore_barrier()

        rows = tuple(sc[f"rows{b}"] for b in range(_NBUF))
        gsem = tuple(sc[f"gsem{b}"] for b in range(_NBUF))
        ssem = tuple(sc[f"ssem{b}"] for b in range(_NBUF))

        def round_body(r, carry):
            pltpu.sync_copy(src_hbm.at[wid, pl.ds(r * _RPT, _RPT)],
                            sc["src_v"])
            pltpu.sync_copy(dst_hbm.at[wid, pl.ds(r * _RPT, _RPT)],
                            sc["dst_v"])

            for i in range(_NBUF):
                pltpu.async_copy(table_hbm.at[sc["src_v"].at[i]],
                                 rows[i], gsem[i])
            for i in range(_RPT):
                p = i % _NBUF
                pltpu.make_async_copy(table_hbm.at[sc["src_v"].at[i]],
                                      rows[p], gsem[p]).wait()
                pltpu.async_copy(rows[p],
                                 sc["acc_sh"].at[sc["dst_v"].at[i]],
                                 ssem[p], add=True).wait()
                if i + _NBUF < _RPT:
                    pltpu.async_copy(
                        table_hbm.at[sc["src_v"].at[i + _NBUF]],
                        rows[p], gsem[p])
            return carry
        lax.fori_loop(0, nround, round_body, 0)

        plsc.subcore_barrier()

        pltpu.sync_copy(
            sc["acc_sh"].at[pl.ds(s * rows_per_tile, rows_per_tile)],
            acc_out.at[c, pl.ds(s * rows_per_tile, rows_per_tile)])

    return pl.kernel(body, out_type=out_type, mesh=mesh,
                     scratch_types=scratch)(table, src2d, dst2d)


def _sc_degree(dst2d, nround):
    mesh = plsc.VectorSubcoreMesh(core_axis_name="c", subcore_axis_name="s")
    rows_per_tile = _NROWS // _NS

    out_type = jax.ShapeDtypeStruct((_NC, _NROWS, 128), jnp.float32)

    scratch = dict(
        dst_v=pltpu.VMEM((_RPT, _C), jnp.int32),
        ones_v=pltpu.VMEM((_C, 128), jnp.float32),
        deg_sh=pltpu.VMEM_SHARED((_NROWS, 128), jnp.float32),
        ssem=pltpu.SemaphoreType.DMA,
    )

    def body(dst_hbm, deg_out, **sc):
        c = lax.axis_index("c")
        s = lax.axis_index("s")
        wid = s * _NC + c

        zero16 = jnp.zeros((16,), jnp.float32)
        ones16 = jnp.ones((16,), jnp.float32)

        def fill_zero(i, carry):
            for j in range(8):
                sc["ones_v"][i, pl.ds(j * 16, 16)] = zero16
            return carry
        lax.fori_loop(0, _C, fill_zero, 0)

        for k in range(4):
            pltpu.sync_copy(
                sc["ones_v"],
                sc["deg_sh"].at[pl.ds(s * rows_per_tile + k * 128, 128)])
        pltpu.sync_copy(
            sc["ones_v"].at[pl.ds(0, 120)],
            sc["deg_sh"].at[pl.ds(s * rows_per_tile + 512, 120)])

        def fill_ones(i, carry):
            for j in range(8):
                sc["ones_v"][i, pl.ds(j * 16, 16)] = ones16
            return carry
        lax.fori_loop(0, _C, fill_ones, 0)

        plsc.subcore_barrier()

        def round_body(r, carry):
            pltpu.sync_copy(dst_hbm.at[wid, pl.ds(r * _RPT, _RPT)],
                            sc["dst_v"])

            def fire(i, carry2):
                pltpu.async_copy(sc["ones_v"],
                                 sc["deg_sh"].at[sc["dst_v"].at[i]],
                                 sc["ssem"], add=True)
                return carry2
            lax.fori_loop(0, _RPT, fire, 0)

            def drain(i, carry2):
                pltpu.make_async_copy(sc["ones_v"],
                                      sc["deg_sh"].at[sc["dst_v"].at[i]],
                                      sc["ssem"]).wait()
                return carry2
            lax.fori_loop(0, _RPT, drain, 0)
            return carry
        lax.fori_loop(0, nround, round_body, 0)

        plsc.subcore_barrier()

        pltpu.sync_copy(
            sc["deg_sh"].at[pl.ds(s * rows_per_tile, rows_per_tile)],
            deg_out.at[c, pl.ds(s * rows_per_tile, rows_per_tile)])

    return pl.kernel(body, out_type=out_type, mesh=mesh,
                     scratch_types=scratch)(dst2d)


def _tc_proj(x, w_l, w_r, b):
    def body(x_ref, wl_ref, wr_ref, b_ref, y_ref, z_ref):
        xb = x_ref[...]
        y_ref[...] = jnp.dot(xb, wl_ref[...],
                             preferred_element_type=jnp.float32)
        z_ref[...] = jnp.dot(xb, wr_ref[...],
                             preferred_element_type=jnp.float32) + b_ref[...]

    return pl.pallas_call(
        body,
        grid=(_NBLK,),
        in_specs=[
            pl.BlockSpec((128, 128), lambda i: (i, 0)),
            pl.BlockSpec((128, 128), lambda i: (0, 0)),
            pl.BlockSpec((128, 128), lambda i: (0, 0)),
            pl.BlockSpec((1, 128), lambda i: (0, 0)),
        ],
        out_specs=[pl.BlockSpec((128, 128), lambda i: (i, 0))] * 2,
        out_shape=[jax.ShapeDtypeStruct((_NROWS, 128), jnp.float32)] * 2,
    )(x, w_l, w_r, b.reshape(1, 128))


def _tc_combine_proj(acc, deg, z1, w_l, w_r, b):
    def body(acc_ref, deg_ref, z1_ref, wl_ref, wr_ref, b_ref,
             y2_ref, z2_ref, rdeg_ref):
        a = acc_ref[0] + acc_ref[1]
        d = (deg_ref[0] + deg_ref[1])[:, 0:1]
        r = 1.0 / jnp.maximum(d, 1.0)
        h = jnp.maximum(a * r + z1_ref[...], 0.0)
        y2_ref[...] = jnp.dot(h, wl_ref[...],
                              preferred_element_type=jnp.float32)
        z2_ref[...] = jnp.dot(h, wr_ref[...],
                              preferred_element_type=jnp.float32) + b_ref[...]
        rdeg_ref[...] = r

    return pl.pallas_call(
        body,
        grid=(_NBLK,),
        in_specs=[
            pl.BlockSpec((_NC, 128, 128), lambda i: (0, i, 0)),
            pl.BlockSpec((_NC, 128, 128), lambda i: (0, i, 0)),
            pl.BlockSpec((128, 128), lambda i: (i, 0)),
            pl.BlockSpec((128, 128), lambda i: (0, 0)),
            pl.BlockSpec((128, 128), lambda i: (0, 0)),
            pl.BlockSpec((1, 128), lambda i: (0, 0)),
        ],
        out_specs=[
            pl.BlockSpec((128, 128), lambda i: (i, 0)),
            pl.BlockSpec((128, 128), lambda i: (i, 0)),
            pl.BlockSpec((128, 1), lambda i: (i, 0)),
        ],
        out_shape=[
            jax.ShapeDtypeStruct((_NROWS, 128), jnp.float32),
            jax.ShapeDtypeStruct((_NROWS, 128), jnp.float32),
            jax.ShapeDtypeStruct((_NROWS, 1), jnp.float32),
        ],
    )(acc, deg, z1, w_l, w_r, b.reshape(1, 128))


def _tc_final(acc, rdeg, z2):
    def body(acc_ref, rdeg_ref, z2_ref, o_ref):
        a = acc_ref[0] + acc_ref[1]
        o_ref[...] = a * rdeg_ref[...] + z2_ref[...]

    return pl.pallas_call(
        body,
        grid=(_NBLK,),
        in_specs=[
            pl.BlockSpec((_NC, 128, 128), lambda i: (0, i, 0)),
            pl.BlockSpec((128, 1), lambda i: (i, 0)),
            pl.BlockSpec((128, 128), lambda i: (i, 0)),
        ],
        out_specs=pl.BlockSpec((128, 128), lambda i: (i, 0)),
        out_shape=jax.ShapeDtypeStruct((_NROWS, 128), jnp.float32),
    )(acc, rdeg, z2)


def kernel(x, edge_index, W1_l, W1_r, b1, W2_l, W2_r, b2):
    E = edge_index.shape[1]
    epr = _NW * _C * _RPT
    e_pad = ((E + epr - 1) // epr) * epr
    nround_d = e_pad // epr
    nround = nround_d

    src = jnp.pad(edge_index[0], (0, e_pad - E))
    dst = jnp.pad(edge_index[1], (0, e_pad - E), constant_values=_N)
    src2d = src.reshape(_NW, nround * _RPT, _C)
    dst2d = dst.reshape(_NW, nround * _RPT, _C)
    dst2deg = dst.reshape(_NW, nround_d * _RPT, _C)

    xp = jnp.pad(x, ((0, _NROWS - _N), (0, 0)))

    y1, z1 = _tc_proj(xp, W1_l, W1_r, b1)
    acc1 = _sc_aggregate(y1, src2d, dst2d, nround)
    deg = _sc_degree(dst2deg, nround_d)
    y2, z2, rdeg = _tc_combine_proj(acc1, deg, z1, W2_l, W2_r, b2)

    acc2 = _sc_aggregate(y2, src2d, dst2d, nround)
    out = _tc_final(acc2, rdeg, z2)
    return out[:_N]
